# baseline (speedup 1.0000x reference)
import jax
import jax.numpy as jnp
from jax import lax
from jax.experimental import pallas as pl
from jax.experimental.pallas import tpu as pltpu
from jax.experimental.pallas import tpu_sc as plsc

VOCAB = 1000000
EMBED_DIM = 64
BATCH = 16384
HIST = 20

_NC = 2
_NS = 16
_NW = _NC * _NS

_TOTAL = BATCH * HIST
_PER_W = _TOTAL // _NW
_CHUNK = 128
_NCHUNK = _PER_W // _CHUNK
_NBUF = 4
_NGRP = _NCHUNK // _NBUF


def _body(idx_hbm, table_hbm, out_hbm, idx_v, bufs, gsems, osems):
    wid = lax.axis_index("s") * _NC + lax.axis_index("c")
    base = wid * _PER_W

    pltpu.sync_copy(idx_hbm.at[wid], idx_v)

    def gather(j, b):
        return pltpu.make_async_copy(
            table_hbm.at[idx_v.at[j]], bufs.at[b], gsems.at[b])

    def put(j, b):
        return pltpu.make_async_copy(
            bufs.at[b], out_hbm.at[pl.ds(base + j * _CHUNK, _CHUNK)],
            osems.at[b])

    for b in range(_NBUF):
        gather(b, b).start()

    def group(g, carry):
        for b in range(_NBUF):
            j = g * _NBUF + b
            gather(j, b).wait()
            put(j, b).start()
            put(j, b).wait()
            gather(j + _NBUF, b).start()
        return carry

    lax.fori_loop(0, _NGRP - 1, group, 0)

    for b in range(_NBUF):
        j = (_NGRP - 1) * _NBUF + b
        gather(j, b).wait()
        put(j, b).start()
    for b in range(_NBUF):
        put((_NGRP - 1) * _NBUF + b, b).wait()


@jax.jit
def _lookup(idx, table):
    mesh = plsc.VectorSubcoreMesh(core_axis_name="c", subcore_axis_name="s")
    k = pl.kernel(
        _body,
        out_type=jax.ShapeDtypeStruct((_TOTAL, EMBED_DIM), jnp.float32),
        mesh=mesh,
        scratch_types=dict(
            idx_v=pltpu.VMEM((_NCHUNK, _CHUNK), jnp.int32),
            bufs=pltpu.VMEM((_NBUF, _CHUNK, EMBED_DIM), jnp.float32),
            gsems=pltpu.SemaphoreType.DMA((_NBUF,)),
            osems=pltpu.SemaphoreType.DMA((_NBUF,)),
        ),
        compiler_params=pltpu.CompilerParams(use_tc_tiling_on_sc=False),
    )
    return k(idx, table)


def kernel(id, W_center, W_context):
    idx = id.astype(jnp.int32).reshape(_NW, _NCHUNK, _CHUNK)
    out = _lookup(idx, W_center)
    return out.reshape(BATCH, HIST, EMBED_DIM)


# speedup vs baseline: 1.0006x; 1.0006x over previous
import jax
import jax.numpy as jnp
from jax import lax
from jax.experimental import pallas as pl
from jax.experimental.pallas import tpu as pltpu
from jax.experimental.pallas import tpu_sc as plsc

VOCAB = 1000000
EMBED_DIM = 64
BATCH = 16384
HIST = 20

_NC = 2
_NS = 16
_NW = _NC * _NS

_TOTAL = BATCH * HIST
_PER_W = _TOTAL // _NW
_CHUNK = 128
_NCHUNK = _PER_W // _CHUNK
_NBUF = 6
_LAG = 3


def _body(idx_hbm, table_hbm, out_hbm, idx_v, bufs, gsems, osems):
    wid = lax.axis_index("s") * _NC + lax.axis_index("c")
    base = wid * _PER_W

    pltpu.sync_copy(idx_hbm.at[wid], idx_v)

    def gather(j, b):
        return pltpu.make_async_copy(
            table_hbm.at[idx_v.at[j]], bufs.at[b], gsems.at[b])

    def put(j, b):
        return pltpu.make_async_copy(
            bufs.at[b], out_hbm.at[pl.ds(base + j * _CHUNK, _CHUNK)],
            osems.at[b])

    for t in range(_NBUF):
        gather(t, t % _NBUF).start()
        s = t - _LAG
        if s >= 0:
            gather(s, s % _NBUF).wait()
            put(s, s % _NBUF).start()

    def step(t, carry):
        b = t % _NBUF
        put(t - _NBUF, b).wait()
        gather(t, b).start()
        s = t - _LAG
        bs = s % _NBUF
        gather(s, bs).wait()
        put(s, bs).start()
        return carry

    lax.fori_loop(_NBUF, _NCHUNK, step, 0)

    for s in range(_NCHUNK - _LAG, _NCHUNK):
        gather(s, s % _NBUF).wait()
        put(s, s % _NBUF).start()
    for s in range(_NCHUNK - _NBUF, _NCHUNK):
        put(s, s % _NBUF).wait()


@jax.jit
def _lookup(idx, table):
    mesh = plsc.VectorSubcoreMesh(core_axis_name="c", subcore_axis_name="s")
    k = pl.kernel(
        _body,
        out_type=jax.ShapeDtypeStruct((_TOTAL, EMBED_DIM), jnp.float32),
        mesh=mesh,
        scratch_types=dict(
            idx_v=pltpu.VMEM((_NCHUNK, _CHUNK), jnp.int32),
            bufs=pltpu.VMEM((_NBUF, _CHUNK, EMBED_DIM), jnp.float32),
            gsems=pltpu.SemaphoreType.DMA((_NBUF,)),
            osems=pltpu.SemaphoreType.DMA((_NBUF,)),
        ),
        compiler_params=pltpu.CompilerParams(use_tc_tiling_on_sc=False),
    )
    return k(idx, table)


def kernel(id, W_center, W_context):
    idx = id.astype(jnp.int32).reshape(_NW, _NCHUNK, _CHUNK)
    out = _lookup(idx, W_center)
    return out.reshape(BATCH, HIST, EMBED_DIM)
